# baseline (speedup 1.0000x reference)
import jax
import jax.numpy as jnp
from jax import lax
from jax.experimental import pallas as pl
from jax.experimental.pallas import tpu as pltpu
from jax.experimental.pallas import tpu_sc as plsc

_ALT_VALUES = (150, 200, 250, 300)
_NUM_PARTS = 16
_NUM_ROWS = 5
_BATCH = 16384
_NC, _NS = 2, 16
_NW = _NC * _NS
_BPW = _BATCH // _NW
_GROUPS = _BPW // 16
_FILL_BLOCK = 2048


def _tc_prefill_kernel(att_ref, temp_ref, table_ref, fill_ref):
    t = jnp.maximum(jnp.abs(temp_ref[0, 0]), jnp.float32(0.1))
    w = att_ref[...] / t
    m = jnp.max(w, axis=-1, keepdims=True)
    e = jnp.exp(w - m)
    sm = e / jnp.sum(e, axis=-1, keepdims=True)

    @pl.when(pl.program_id(0) == 0)
    def _():
        table_ref[...] = sm

    fill_ref[...] = jnp.broadcast_to(sm[_NUM_ROWS - 1:_NUM_ROWS, :],
                                     fill_ref.shape)


def _tc_prefill(attention, temp):
    return pl.pallas_call(
        _tc_prefill_kernel,
        grid=(_BATCH // _FILL_BLOCK,),
        out_shape=(
            jax.ShapeDtypeStruct((_NUM_ROWS, _NUM_PARTS), jnp.float32),
            jax.ShapeDtypeStruct((_BATCH, _NUM_PARTS), jnp.float32),
        ),
        in_specs=[
            pl.BlockSpec((_NUM_ROWS, _NUM_PARTS), lambda i: (0, 0)),
            pl.BlockSpec(memory_space=pltpu.SMEM),
        ],
        out_specs=(
            pl.BlockSpec((_NUM_ROWS, _NUM_PARTS), lambda i: (0, 0)),
            pl.BlockSpec((_FILL_BLOCK, _NUM_PARTS), lambda i: (i, 0)),
        ),
    )(attention, temp.reshape(1, 1))


def _sc_fixup_kernel(table_hbm, alt_hbm, out_hbm, table_v, alt_v, sem):
    wid = lax.axis_index("s") * _NC + lax.axis_index("c")
    base = wid * _BPW
    pltpu.sync_copy(table_hbm, table_v)
    pltpu.sync_copy(alt_hbm.at[pl.ds(base, _BPW)], alt_v)

    def group_body(gg, cnt):
        a = alt_v[pl.ds(gg * 16, 16)]
        idx = jnp.full((16,), _NUM_ROWS - 1, dtype=jnp.int32)
        for i, v in enumerate(_ALT_VALUES):
            idx = jnp.where(a == jnp.int32(v), jnp.int32(i), idx)
        nhit = jnp.sum(jnp.where(idx != _NUM_ROWS - 1, 1, 0).astype(jnp.int32))

        @pl.when(nhit > 0)
        def _fixup():
            for k in range(16):
                ik = idx[k]

                @pl.when(ik != _NUM_ROWS - 1)
                def _one(ik=ik, k=k):
                    pltpu.async_copy(
                        table_v.at[ik],
                        out_hbm.at[base + gg * 16 + k],
                        sem,
                    )
        return cnt + nhit

    total = lax.fori_loop(0, _GROUPS, group_body, jnp.int32(0))

    def drain_body(i, carry):
        pltpu.make_async_copy(
            table_v.at[0], out_hbm.at[base], sem).wait()
        return carry

    lax.fori_loop(0, total, drain_body, 0)


def kernel(altitudes, attention, temp):
    table, filled = _tc_prefill(attention, temp)
    out_ref = jax.new_ref(filled)
    mesh = plsc.VectorSubcoreMesh(core_axis_name="c", subcore_axis_name="s")
    run = pl.kernel(
        _sc_fixup_kernel,
        out_type=(),
        mesh=mesh,
        compiler_params=pltpu.CompilerParams(
            use_tc_tiling_on_sc=True, needs_layout_passes=False),
        scratch_types=[
            pltpu.VMEM((_NUM_ROWS, _NUM_PARTS), jnp.float32),
            pltpu.VMEM((_BPW,), jnp.int32),
            pltpu.SemaphoreType.DMA,
        ],
    )
    run(table, altitudes, out_ref)
    return jax.freeze(out_ref)
